# Initial kernel scaffold; baseline (speedup 1.0000x reference)
import jax
import jax.numpy as jnp
from jax.experimental import pallas as pl


def kernel(feat_user, feat_item, edge_ui, edge_iu, W_proj, b_proj, ui_W1, ui_b1, ui_g, ui_bb, ui_W2, ui_b2, ret_W1, ret_b1, ret_g, ret_bb, ret_W2, ret_b2, eps_ui, eps_ret):
    raise NotImplementedError("write your pallas kernel here")


# speedup vs baseline: 5.1270x; 5.1270x over previous
import functools

import jax
import jax.numpy as jnp
from jax import lax
from jax.experimental import pallas as pl
from jax.experimental.pallas import tpu as pltpu
from jax.experimental.pallas import tpu_sc as plsc

_NC = 2
_NS = 16
_NW = _NC * _NS
_K = 80


def _segment_sum_sc(table, src, dst, n_out):
    e = src.shape[0]
    d = table.shape[1]
    per_w = e // _NW
    n_chunks = per_w // _K
    assert per_w * _NW == e and n_chunks * _K == per_w
    rows_per_tile = (n_out // _NS) // 8 * 8
    tail_rows = n_out - rows_per_tile * _NS
    assert 0 <= tail_rows <= rows_per_tile or rows_per_tile == 0

    mesh = plsc.VectorSubcoreMesh(core_axis_name="c", subcore_axis_name="s")

    @functools.partial(
        pl.kernel,
        mesh=mesh,
        out_type=jax.ShapeDtypeStruct((_NC, n_out, d), jnp.float32),
        scratch_types=[
            pltpu.VMEM_SHARED((n_out, d), jnp.float32),
            pltpu.VMEM((_K,), jnp.int32),
            pltpu.VMEM((_K,), jnp.int32),
            pltpu.VMEM((_K, d), jnp.float32),
            pltpu.SemaphoreType.DMA,
        ],
    )
    def seg_kernel(table_h, src_h, dst_h, zero_h, out_h, acc, sidx, didx,
                   rows, sem):
        cid = lax.axis_index("c")
        sid = lax.axis_index("s")
        wid = sid * _NC + cid
        pltpu.sync_copy(zero_h,
                        acc.at[pl.ds(sid * rows_per_tile, rows_per_tile)])
        if tail_rows:
            @pl.when(sid == _NS - 1)
            def _():
                pltpu.sync_copy(
                    zero_h.at[pl.ds(0, tail_rows)],
                    acc.at[pl.ds(_NS * rows_per_tile, tail_rows)])
        plsc.subcore_barrier()

        ebase = wid * per_w

        def body(j, carry):
            base = ebase + j * _K
            pltpu.sync_copy(src_h.at[pl.ds(base, _K)], sidx)
            pltpu.sync_copy(dst_h.at[pl.ds(base, _K)], didx)
            pltpu.async_copy(table_h.at[sidx], rows, sem).wait()
            pltpu.sync_copy(rows, acc.at[didx], add=True)
            return carry

        lax.fori_loop(0, n_chunks, body, 0)
        plsc.subcore_barrier()
        pltpu.sync_copy(acc.at[pl.ds(sid * rows_per_tile, rows_per_tile)],
                        out_h.at[cid, pl.ds(sid * rows_per_tile,
                                            rows_per_tile)])
        if tail_rows:
            @pl.when(sid == _NS - 1)
            def _():
                pltpu.sync_copy(acc.at[pl.ds(_NS * rows_per_tile, tail_rows)],
                                out_h.at[cid, pl.ds(_NS * rows_per_tile,
                                                    tail_rows)])

    zeros = jnp.zeros((max(rows_per_tile, tail_rows), d), jnp.float32)
    return seg_kernel(table, src, dst, zeros)


def _gin_mlp(x, partials, w1, b1, g, bb, w2, b2, eps):
    n, d = x.shape

    def body(x_ref, p_ref, w1_ref, b1_ref, g_ref, bb_ref, w2_ref, b2_ref,
             eps_ref, out_ref):
        xin = (1.0 + eps_ref[0]) * x_ref[...] + p_ref[0] + p_ref[1]
        h = lax.dot_general(xin, w1_ref[...], (((1,), (1,)), ((), ())),
                            preferred_element_type=jnp.float32) + b1_ref[...]
        mu = jnp.mean(h, axis=0, keepdims=True)
        var = jnp.mean((h - mu) ** 2, axis=0, keepdims=True)
        hn = (h - mu) * lax.rsqrt(var + 1e-5) * g_ref[...] + bb_ref[...]
        hr = jnp.maximum(hn, 0.0)
        out_ref[...] = lax.dot_general(hr, w2_ref[...], (((1,), (1,)), ((), ())),
                                       preferred_element_type=jnp.float32
                                       ) + b2_ref[...]

    vspec = pl.BlockSpec(memory_space=pltpu.MemorySpace.VMEM)
    sspec = pl.BlockSpec(memory_space=pltpu.MemorySpace.SMEM)
    return pl.pallas_call(
        body,
        out_shape=jax.ShapeDtypeStruct((n, d), jnp.float32),
        in_specs=[vspec] * 8 + [sspec],
        out_specs=vspec,
    )(x, partials, w1, b1, g, bb, w2, b2, eps)


def _proj_gin_mlp(x, wp, bp, partials, w1, b1, g, bb, w2, b2, eps):
    n, d = x.shape

    def body(x_ref, wp_ref, bp_ref, p_ref, w1_ref, b1_ref, g_ref, bb_ref,
             w2_ref, b2_ref, eps_ref, out_ref):
        xp = lax.dot_general(x_ref[...], wp_ref[...], (((1,), (1,)), ((), ())),
                             preferred_element_type=jnp.float32) + bp_ref[...]
        xin = (1.0 + eps_ref[0]) * xp + p_ref[0] + p_ref[1]
        h = lax.dot_general(xin, w1_ref[...], (((1,), (1,)), ((), ())),
                            preferred_element_type=jnp.float32) + b1_ref[...]
        mu = jnp.mean(h, axis=0, keepdims=True)
        var = jnp.mean((h - mu) ** 2, axis=0, keepdims=True)
        hn = (h - mu) * lax.rsqrt(var + 1e-5) * g_ref[...] + bb_ref[...]
        hr = jnp.maximum(hn, 0.0)
        out_ref[...] = lax.dot_general(hr, w2_ref[...], (((1,), (1,)), ((), ())),
                                       preferred_element_type=jnp.float32
                                       ) + b2_ref[...]

    vspec = pl.BlockSpec(memory_space=pltpu.MemorySpace.VMEM)
    sspec = pl.BlockSpec(memory_space=pltpu.MemorySpace.SMEM)
    return pl.pallas_call(
        body,
        out_shape=jax.ShapeDtypeStruct((n, d), jnp.float32),
        in_specs=[vspec] * 10 + [sspec],
        out_specs=vspec,
    )(x, wp, bp, partials, w1, b1, g, bb, w2, b2, eps)


def kernel(feat_user, feat_item, edge_ui, edge_iu, W_proj, b_proj,
           ui_W1, ui_b1, ui_g, ui_bb, ui_W2, ui_b2,
           ret_W1, ret_b1, ret_g, ret_bb, ret_W2, ret_b2,
           eps_ui, eps_ret):
    n = feat_user.shape[0]
    p_item = _segment_sum_sc(feat_user, edge_ui[0], edge_ui[1], n)
    h_item = _gin_mlp(feat_item, p_item, ui_W1, ui_b1, ui_g, ui_bb,
                      ui_W2, ui_b2, eps_ui)
    p_user = _segment_sum_sc(h_item, edge_iu[0], edge_iu[1], n)
    h_user = _proj_gin_mlp(feat_user, W_proj, b_proj, p_user, ret_W1, ret_b1,
                           ret_g, ret_bb, ret_W2, ret_b2, eps_ret)
    return (h_user, h_item)


# speedup vs baseline: 11.7242x; 2.2868x over previous
import functools

import jax
import jax.numpy as jnp
from jax import lax
from jax.experimental import pallas as pl
from jax.experimental.pallas import tpu as pltpu
from jax.experimental.pallas import tpu_sc as plsc

_NC = 2
_NS = 16
_NW = _NC * _NS
_K = 80


def _segment_sum_sc(table, src, dst, n_out):
    e = src.shape[0]
    d = table.shape[1]
    per_w = e // _NW
    n_chunks = per_w // _K
    assert per_w * _NW == e and n_chunks * _K == per_w
    rows_per_tile = (n_out // _NS) // 8 * 8
    tail_rows = n_out - rows_per_tile * _NS
    assert 0 <= tail_rows <= rows_per_tile or rows_per_tile == 0

    mesh = plsc.VectorSubcoreMesh(core_axis_name="c", subcore_axis_name="s")

    @functools.partial(
        pl.kernel,
        mesh=mesh,
        out_type=jax.ShapeDtypeStruct((_NC, n_out, d), jnp.float32),
        scratch_types=[
            pltpu.VMEM_SHARED((n_out, d), jnp.float32),
            pltpu.VMEM((n_chunks, _K), jnp.int32),
            pltpu.VMEM((_K,), jnp.int32),
            pltpu.VMEM((_K,), jnp.int32),
            pltpu.VMEM((_K, d), jnp.float32),
            pltpu.VMEM((_K, d), jnp.float32),
            pltpu.SemaphoreType.DMA,
            pltpu.SemaphoreType.DMA,
            pltpu.SemaphoreType.DMA,
            pltpu.SemaphoreType.DMA,
        ],
    )
    def seg_kernel(table_h, src_h, dst_h, zero_h, out_h, acc, sidx,
                   didx0, didx1, rows0, rows1, sem0, sem1, dsem0, dsem1):
        cid = lax.axis_index("c")
        sid = lax.axis_index("s")
        wid = sid * _NC + cid
        icp0 = pltpu.async_copy(src_h.at[wid], sidx, sem0)
        pltpu.sync_copy(zero_h,
                        acc.at[pl.ds(sid * rows_per_tile, rows_per_tile)])
        if tail_rows:
            @pl.when(sid == _NS - 1)
            def _():
                pltpu.sync_copy(
                    zero_h.at[pl.ds(0, tail_rows)],
                    acc.at[pl.ds(_NS * rows_per_tile, tail_rows)])
        icp0.wait()
        plsc.subcore_barrier()

        pltpu.async_copy(table_h.at[sidx.at[0]], rows0, sem0)
        pltpu.async_copy(dst_h.at[pl.ds(wid * per_w, _K)], didx0, dsem0)

        def body(i, carry):
            c0 = 2 * i
            g1 = pltpu.async_copy(table_h.at[sidx.at[c0 + 1]], rows1, sem1)
            d1 = pltpu.async_copy(
                dst_h.at[pl.ds(wid * per_w + (c0 + 1) * _K, _K)], didx1, dsem1)
            pltpu.make_async_copy(table_h.at[sidx.at[c0]], rows0, sem0).wait()
            pltpu.make_async_copy(
                dst_h.at[pl.ds(wid * per_w + c0 * _K, _K)], didx0, dsem0).wait()
            pltpu.sync_copy(rows0, acc.at[didx0], add=True)
            pltpu.async_copy(table_h.at[sidx.at[c0 + 2]], rows0, sem0)
            pltpu.async_copy(
                dst_h.at[pl.ds(wid * per_w + (c0 + 2) * _K, _K)], didx0, dsem0)
            g1.wait()
            d1.wait()
            pltpu.sync_copy(rows1, acc.at[didx1], add=True)
            return carry

        lax.fori_loop(0, (n_chunks - 1) // 2, body, 0)
        pltpu.make_async_copy(table_h.at[sidx.at[n_chunks - 1]], rows0,
                              sem0).wait()
        pltpu.make_async_copy(
            dst_h.at[pl.ds(wid * per_w + (n_chunks - 1) * _K, _K)], didx0,
            dsem0).wait()
        pltpu.sync_copy(rows0, acc.at[didx0], add=True)
        plsc.subcore_barrier()
        pltpu.sync_copy(acc.at[pl.ds(sid * rows_per_tile, rows_per_tile)],
                        out_h.at[cid, pl.ds(sid * rows_per_tile,
                                            rows_per_tile)])
        if tail_rows:
            @pl.when(sid == _NS - 1)
            def _():
                pltpu.sync_copy(acc.at[pl.ds(_NS * rows_per_tile, tail_rows)],
                                out_h.at[cid, pl.ds(_NS * rows_per_tile,
                                                    tail_rows)])

    zeros = jnp.zeros((max(rows_per_tile, tail_rows), d), jnp.float32)
    src3 = src.reshape(_NW, n_chunks, _K)
    return seg_kernel(table, src3, dst, zeros)


def _gin_mlp(x, partials, w1, b1, g, bb, w2, b2, eps):
    n, d = x.shape

    def body(x_ref, p_ref, w1_ref, b1_ref, g_ref, bb_ref, w2_ref, b2_ref,
             eps_ref, out_ref):
        xin = (1.0 + eps_ref[0]) * x_ref[...] + p_ref[0] + p_ref[1]
        h = lax.dot_general(xin, w1_ref[...], (((1,), (1,)), ((), ())),
                            preferred_element_type=jnp.float32) + b1_ref[...]
        mu = jnp.mean(h, axis=0, keepdims=True)
        var = jnp.mean((h - mu) ** 2, axis=0, keepdims=True)
        hn = (h - mu) * lax.rsqrt(var + 1e-5) * g_ref[...] + bb_ref[...]
        hr = jnp.maximum(hn, 0.0)
        out_ref[...] = lax.dot_general(hr, w2_ref[...], (((1,), (1,)), ((), ())),
                                       preferred_element_type=jnp.float32
                                       ) + b2_ref[...]

    vspec = pl.BlockSpec(memory_space=pltpu.MemorySpace.VMEM)
    sspec = pl.BlockSpec(memory_space=pltpu.MemorySpace.SMEM)
    return pl.pallas_call(
        body,
        out_shape=jax.ShapeDtypeStruct((n, d), jnp.float32),
        in_specs=[vspec] * 8 + [sspec],
        out_specs=vspec,
    )(x, partials, w1, b1, g, bb, w2, b2, eps)


def _proj_gin_mlp(x, wp, bp, partials, w1, b1, g, bb, w2, b2, eps):
    n, d = x.shape

    def body(x_ref, wp_ref, bp_ref, p_ref, w1_ref, b1_ref, g_ref, bb_ref,
             w2_ref, b2_ref, eps_ref, out_ref):
        xp = lax.dot_general(x_ref[...], wp_ref[...], (((1,), (1,)), ((), ())),
                             preferred_element_type=jnp.float32) + bp_ref[...]
        xin = (1.0 + eps_ref[0]) * xp + p_ref[0] + p_ref[1]
        h = lax.dot_general(xin, w1_ref[...], (((1,), (1,)), ((), ())),
                            preferred_element_type=jnp.float32) + b1_ref[...]
        mu = jnp.mean(h, axis=0, keepdims=True)
        var = jnp.mean((h - mu) ** 2, axis=0, keepdims=True)
        hn = (h - mu) * lax.rsqrt(var + 1e-5) * g_ref[...] + bb_ref[...]
        hr = jnp.maximum(hn, 0.0)
        out_ref[...] = lax.dot_general(hr, w2_ref[...], (((1,), (1,)), ((), ())),
                                       preferred_element_type=jnp.float32
                                       ) + b2_ref[...]

    vspec = pl.BlockSpec(memory_space=pltpu.MemorySpace.VMEM)
    sspec = pl.BlockSpec(memory_space=pltpu.MemorySpace.SMEM)
    return pl.pallas_call(
        body,
        out_shape=jax.ShapeDtypeStruct((n, d), jnp.float32),
        in_specs=[vspec] * 10 + [sspec],
        out_specs=vspec,
    )(x, wp, bp, partials, w1, b1, g, bb, w2, b2, eps)


def kernel(feat_user, feat_item, edge_ui, edge_iu, W_proj, b_proj,
           ui_W1, ui_b1, ui_g, ui_bb, ui_W2, ui_b2,
           ret_W1, ret_b1, ret_g, ret_bb, ret_W2, ret_b2,
           eps_ui, eps_ret):
    n = feat_user.shape[0]
    p_item = _segment_sum_sc(feat_user, edge_ui[0], edge_ui[1], n)
    h_item = _gin_mlp(feat_item, p_item, ui_W1, ui_b1, ui_g, ui_bb,
                      ui_W2, ui_b2, eps_ui)
    p_user = _segment_sum_sc(h_item, edge_iu[0], edge_iu[1], n)
    h_user = _proj_gin_mlp(feat_user, W_proj, b_proj, p_user, ret_W1, ret_b1,
                           ret_g, ret_bb, ret_W2, ret_b2, eps_ret)
    return (h_user, h_item)


# speedup vs baseline: 13.3753x; 1.1408x over previous
import functools

import jax
import jax.numpy as jnp
from jax import lax
from jax.experimental import pallas as pl
from jax.experimental.pallas import tpu as pltpu
from jax.experimental.pallas import tpu_sc as plsc

_NC = 2
_NS = 16
_NW = _NC * _NS
_K = 80


def _segment_sum_sc(table, src, dst, n_out):
    e = src.shape[0]
    d = table.shape[1]
    per_w = e // _NW
    n_chunks = per_w // _K
    assert per_w * _NW == e and n_chunks * _K == per_w
    rows_per_tile = (n_out // _NS) // 8 * 8
    tail_rows = n_out - rows_per_tile * _NS
    assert 0 <= tail_rows <= rows_per_tile or rows_per_tile == 0

    mesh = plsc.VectorSubcoreMesh(core_axis_name="c", subcore_axis_name="s")

    @functools.partial(
        pl.kernel,
        mesh=mesh,
        out_type=jax.ShapeDtypeStruct((_NC, n_out, d), jnp.float32),
        scratch_types=[
            pltpu.VMEM_SHARED((n_out, d), jnp.float32),
            pltpu.VMEM((n_chunks, _K), jnp.int32),
            [pltpu.VMEM((_K,), jnp.int32) for _ in range(3)],
            [pltpu.VMEM((_K, d), jnp.float32) for _ in range(3)],
            [pltpu.SemaphoreType.DMA for _ in range(3)],
            [pltpu.SemaphoreType.DMA for _ in range(3)],
            [pltpu.SemaphoreType.DMA for _ in range(3)],
        ],
    )
    def seg_kernel(table_h, src_h, dst_h, zero_h, out_h, acc, sidx,
                   didx, rows, gsem, dsem, ssem):
        cid = lax.axis_index("c")
        sid = lax.axis_index("s")
        wid = sid * _NC + cid
        icp0 = pltpu.async_copy(src_h.at[wid], sidx, gsem[0])
        pltpu.sync_copy(zero_h,
                        acc.at[pl.ds(sid * rows_per_tile, rows_per_tile)])
        if tail_rows:
            @pl.when(sid == _NS - 1)
            def _():
                pltpu.sync_copy(
                    zero_h.at[pl.ds(0, tail_rows)],
                    acc.at[pl.ds(_NS * rows_per_tile, tail_rows)])
        icp0.wait()
        plsc.subcore_barrier()

        ebase = wid * per_w

        def start_fetch(c, b):
            pltpu.async_copy(table_h.at[sidx.at[c]], rows[b], gsem[b])
            pltpu.async_copy(dst_h.at[pl.ds(ebase + c * _K, _K)], didx[b],
                             dsem[b])

        def wait_fetch(c, b):
            pltpu.make_async_copy(table_h.at[sidx.at[c]], rows[b],
                                  gsem[b]).wait()
            pltpu.make_async_copy(dst_h.at[pl.ds(ebase + c * _K, _K)],
                                  didx[b], dsem[b]).wait()

        def wait_scatter(b):
            pltpu.make_async_copy(rows[b], acc.at[didx[b]], ssem[b]).wait()

        start_fetch(0, 0)
        start_fetch(1, 1)
        n_main = (n_chunks - 2) // 3

        def body(i, carry):
            for b in range(3):
                c = 3 * i + b
                wait_fetch(c, b)
                pltpu.async_copy(rows[b], acc.at[didx[b]], ssem[b], add=True)
                b2 = (b + 2) % 3
                if b == 0:
                    @pl.when(i > 0)
                    def _():
                        wait_scatter(b2)
                else:
                    wait_scatter(b2)
                start_fetch(c + 2, b2)
            return carry

        lax.fori_loop(0, n_main, body, 0)
        for t in range(n_chunks - 3 * n_main, 0, -1):
            c = n_chunks - t
            b = c % 3
            wait_fetch(c, b)
            pltpu.async_copy(rows[b], acc.at[didx[b]], ssem[b], add=True)
        for b in range(3):
            wait_scatter(b)
        plsc.subcore_barrier()
        pltpu.sync_copy(acc.at[pl.ds(sid * rows_per_tile, rows_per_tile)],
                        out_h.at[cid, pl.ds(sid * rows_per_tile,
                                            rows_per_tile)])
        if tail_rows:
            @pl.when(sid == _NS - 1)
            def _():
                pltpu.sync_copy(acc.at[pl.ds(_NS * rows_per_tile, tail_rows)],
                                out_h.at[cid, pl.ds(_NS * rows_per_tile,
                                                    tail_rows)])

    zeros = jnp.zeros((max(rows_per_tile, tail_rows), d), jnp.float32)
    src3 = src.reshape(_NW, n_chunks, _K)
    return seg_kernel(table, src3, dst, zeros)


def _gin_mlp(x, partials, w1, b1, g, bb, w2, b2, eps):
    n, d = x.shape

    def body(x_ref, p_ref, w1_ref, b1_ref, g_ref, bb_ref, w2_ref, b2_ref,
             eps_ref, out_ref):
        xin = (1.0 + eps_ref[0]) * x_ref[...] + p_ref[0] + p_ref[1]
        h = lax.dot_general(xin, w1_ref[...], (((1,), (1,)), ((), ())),
                            preferred_element_type=jnp.float32) + b1_ref[...]
        mu = jnp.mean(h, axis=0, keepdims=True)
        var = jnp.mean((h - mu) ** 2, axis=0, keepdims=True)
        hn = (h - mu) * lax.rsqrt(var + 1e-5) * g_ref[...] + bb_ref[...]
        hr = jnp.maximum(hn, 0.0)
        out_ref[...] = lax.dot_general(hr, w2_ref[...], (((1,), (1,)), ((), ())),
                                       preferred_element_type=jnp.float32
                                       ) + b2_ref[...]

    vspec = pl.BlockSpec(memory_space=pltpu.MemorySpace.VMEM)
    sspec = pl.BlockSpec(memory_space=pltpu.MemorySpace.SMEM)
    return pl.pallas_call(
        body,
        out_shape=jax.ShapeDtypeStruct((n, d), jnp.float32),
        in_specs=[vspec] * 8 + [sspec],
        out_specs=vspec,
    )(x, partials, w1, b1, g, bb, w2, b2, eps)


def _proj_gin_mlp(x, wp, bp, partials, w1, b1, g, bb, w2, b2, eps):
    n, d = x.shape

    def body(x_ref, wp_ref, bp_ref, p_ref, w1_ref, b1_ref, g_ref, bb_ref,
             w2_ref, b2_ref, eps_ref, out_ref):
        xp = lax.dot_general(x_ref[...], wp_ref[...], (((1,), (1,)), ((), ())),
                             preferred_element_type=jnp.float32) + bp_ref[...]
        xin = (1.0 + eps_ref[0]) * xp + p_ref[0] + p_ref[1]
        h = lax.dot_general(xin, w1_ref[...], (((1,), (1,)), ((), ())),
                            preferred_element_type=jnp.float32) + b1_ref[...]
        mu = jnp.mean(h, axis=0, keepdims=True)
        var = jnp.mean((h - mu) ** 2, axis=0, keepdims=True)
        hn = (h - mu) * lax.rsqrt(var + 1e-5) * g_ref[...] + bb_ref[...]
        hr = jnp.maximum(hn, 0.0)
        out_ref[...] = lax.dot_general(hr, w2_ref[...], (((1,), (1,)), ((), ())),
                                       preferred_element_type=jnp.float32
                                       ) + b2_ref[...]

    vspec = pl.BlockSpec(memory_space=pltpu.MemorySpace.VMEM)
    sspec = pl.BlockSpec(memory_space=pltpu.MemorySpace.SMEM)
    return pl.pallas_call(
        body,
        out_shape=jax.ShapeDtypeStruct((n, d), jnp.float32),
        in_specs=[vspec] * 10 + [sspec],
        out_specs=vspec,
    )(x, wp, bp, partials, w1, b1, g, bb, w2, b2, eps)


def kernel(feat_user, feat_item, edge_ui, edge_iu, W_proj, b_proj,
           ui_W1, ui_b1, ui_g, ui_bb, ui_W2, ui_b2,
           ret_W1, ret_b1, ret_g, ret_bb, ret_W2, ret_b2,
           eps_ui, eps_ret):
    n = feat_user.shape[0]
    p_item = _segment_sum_sc(feat_user, edge_ui[0], edge_ui[1], n)
    h_item = _gin_mlp(feat_item, p_item, ui_W1, ui_b1, ui_g, ui_bb,
                      ui_W2, ui_b2, eps_ui)
    p_user = _segment_sum_sc(h_item, edge_iu[0], edge_iu[1], n)
    h_user = _proj_gin_mlp(feat_user, W_proj, b_proj, p_user, ret_W1, ret_b1,
                           ret_g, ret_bb, ret_W2, ret_b2, eps_ret)
    return (h_user, h_item)
